# baseline (speedup 1.0000x reference)
import functools

import jax
import jax.numpy as jnp
from jax import lax
from jax.experimental import pallas as pl
from jax.experimental.pallas import tpu as pltpu
from jax.experimental.pallas import tpu_sc as plsc

_K = 3
_G = 16


def _splat(vec, isplat):
    dn = lax.GatherDimensionNumbers(
        offset_dims=(), collapsed_slice_dims=(0,), start_index_map=(0,))
    return lax.gather(vec, isplat[:, None], dn, (1,),
                      mode=lax.GatherScatterMode.PROMISE_IN_BOUNDS)


def _select_body(qT_ref, src_ref, sel_ref, *, S, b_base):
    b = pl.program_id(0) + b_base
    qf = qT_ref[0]
    sf = src_ref[0]
    qx = qf[0:1, :]
    qy = qf[1:2, :]
    qz = qf[2:3, :]
    sx = sf[:, 0:1]
    sy = sf[:, 1:2]
    sz = sf[:, 2:3]
    q2 = qx * qx + qy * qy + qz * qz
    s2 = sx * sx + sy * sy + sz * sz
    m = jnp.dot(sf, qf, preferred_element_type=jnp.float32,
                precision=lax.Precision.DEFAULT)
    d = -2.0 * m
    d = d + q2
    d = d + s2

    iota = lax.broadcasted_iota(jnp.int32, d.shape, 0)
    dd = d
    mins = []
    idxs = []
    for _ in range(_K):
        mk = jnp.min(dd, axis=0, keepdims=True)
        ik = jnp.min(jnp.where(dd <= mk, iota, S), axis=0, keepdims=True)
        dd = jnp.where(iota == ik, jnp.inf, dd)
        mins.append(mk)
        idxs.append(ik)

    recips = [1.0 / (m_ + 1e-8) for m_ in mins]
    norm = (recips[0] + recips[1]) + recips[2]
    gidx = jnp.concatenate(idxs, axis=0) + b * S
    wbits = lax.bitcast_convert_type(
        jnp.concatenate([r / norm for r in recips], axis=0), jnp.int32)
    sel_ref[0] = jnp.concatenate([gidx, wbits], axis=0)


def _select_tc(qT, src, *, nblk, b_base=0):
    B, _, N = qT.shape
    S = src.shape[1]
    grid = (B, N // nblk)
    return pl.pallas_call(
        functools.partial(_select_body, S=S, b_base=b_base),
        grid=grid,
        in_specs=[
            pl.BlockSpec((1, 3, nblk), lambda b, j: (b, 0, j)),
            pl.BlockSpec((1, S, 3), lambda b, j: (b, 0, 0)),
        ],
        out_specs=pl.BlockSpec((1, 6, nblk), lambda b, j: (b, 0, j)),
        out_shape=jax.ShapeDtypeStruct((B, 6, N), jnp.int32),
        compiler_params=pltpu.CompilerParams(
            dimension_semantics=("parallel", "arbitrary"),
        ),
    )(qT, src)


def _interp_sc(table, sel):
    R, D = table.shape
    BN = sel.shape[0] * _G
    info = plsc.get_sparse_core_info()
    NC, NS = info.num_cores, info.num_subcores
    NW = NC * NS
    C = BN // NW
    nsub = C // _G
    npair = nsub // 2
    mesh = plsc.VectorSubcoreMesh(core_axis_name="c", subcore_axis_name="s")

    @functools.partial(
        pl.kernel, mesh=mesh,
        out_type=jax.ShapeDtypeStruct((BN, D), jnp.float32),
        scratch_types=[
            pltpu.VMEM((6 * _G,), jnp.int32),
            pltpu.VMEM((6 * _G,), jnp.int32),
            pltpu.VMEM((_G, D), jnp.float32),
            pltpu.VMEM((_G, D), jnp.float32),
            pltpu.VMEM((_G, D), jnp.float32),
            pltpu.VMEM((_G, D), jnp.float32),
            pltpu.VMEM((_G, D), jnp.float32),
            pltpu.VMEM((_G, D), jnp.float32),
            pltpu.VMEM((_G, D), jnp.float32),
            pltpu.VMEM((_G, D), jnp.float32),
            pltpu.SemaphoreType.DMA,
            pltpu.SemaphoreType.DMA,
        ],
    )
    def k(t_hbm, sel_hbm, out_hbm,
          selA, selB, r0A, r1A, r2A, accA, r0B, r1B, r2B, accB, semA, semB):
        wid = lax.axis_index("s") * NC + lax.axis_index("c")
        base = wid * C
        bufA = (selA, (r0A, r1A, r2A), accA, semA)
        bufB = (selB, (r0B, r1B, r2B), accB, semB)

        def fire(c, buf):
            sel_v, rows, _, sem = buf
            cg = wid * nsub + c
            pltpu.sync_copy(sel_hbm.at[cg], sel_v)
            for kk in range(_K):
                idxk = sel_v[pl.ds(kk * _G, _G)]
                pltpu.async_copy(t_hbm.at[idxk], rows[kk], sem)

        def drain_compute_store(c, buf):
            sel_v, rows, acc, sem = buf
            goff = base + c * _G
            for kk in range(_K):
                idxk = sel_v[pl.ds(kk * _G, _G)]
                pltpu.make_async_copy(t_hbm.at[idxk], rows[kk], sem).wait()
            w0 = lax.bitcast_convert_type(sel_v[pl.ds(3 * _G, _G)], jnp.float32)
            w1 = lax.bitcast_convert_type(sel_v[pl.ds(4 * _G, _G)], jnp.float32)
            w2 = lax.bitcast_convert_type(sel_v[pl.ds(5 * _G, _G)], jnp.float32)
            r0, r1, r2 = rows

            def per_q(i, _):
                isplat = jnp.full((_G,), 0, jnp.int32) + i
                s0 = _splat(w0, isplat)
                s1 = _splat(w1, isplat)
                s2_ = _splat(w2, isplat)
                for v in range(D // _G):
                    sl = pl.ds(v * _G, _G)
                    acc[i, sl] = (r0[i, sl] * s0 + r1[i, sl] * s1) + r2[i, sl] * s2_
                return 0

            lax.fori_loop(0, _G, per_q, 0)
            pltpu.sync_copy(acc, out_hbm.at[pl.ds(goff, _G)])

        fire(0, bufA)

        def pair(p, _):
            cA = 2 * p
            fire(cA + 1, bufB)
            drain_compute_store(cA, bufA)

            @pl.when(cA + 2 < nsub)
            def _():
                fire(cA + 2, bufA)

            drain_compute_store(cA + 1, bufB)
            return 0

        lax.fori_loop(0, npair, pair, 0)

    return k(table, sel)


def _finalize_body(x0_ref, it_ref, *rest, D1):
    out_ref = rest[-1]
    out_ref[0, :D1, :] = x0_ref[0]
    out_ref[0, D1:, :] = jnp.transpose(it_ref[0], (1, 0))


def _finalize_tc(x0, interp, *, nblk, b_base=0, nb=None, out=None):
    B, D1, N = x0.shape
    D2 = interp.shape[2]
    nb = B if nb is None else nb
    args = [x0, interp]
    in_specs = [
        pl.BlockSpec((1, D1, nblk), lambda b, j: (b + b_base, 0, j)),
        pl.BlockSpec((1, nblk, D2), lambda b, j: (b, j, 0)),
    ]
    io_alias = {}
    if out is not None:
        args.append(out)
        in_specs.append(pl.BlockSpec(memory_space=pltpu.MemorySpace.HBM))
        io_alias = {2: 0}
    return pl.pallas_call(
        functools.partial(_finalize_body, D1=D1),
        grid=(nb, N // nblk),
        in_specs=in_specs,
        out_specs=pl.BlockSpec((1, D1 + D2, nblk),
                               lambda b, j: (b + b_base, 0, j)),
        out_shape=jax.ShapeDtypeStruct((B, D1 + D2, N), jnp.float32),
        input_output_aliases=io_alias,
        compiler_params=pltpu.CompilerParams(
            dimension_semantics=("parallel", "arbitrary"),
        ),
    )(*args)


def _flatten_sel(sel):
    B, _, N = sel.shape
    f = jnp.transpose(sel, (1, 0, 2)).reshape(6, (B * N) // _G, _G)
    return jnp.transpose(f, (1, 0, 2)).reshape((B * N) // _G, 6 * _G)


def kernel(xyz0, xyz1, xyz2, x0, x1, x2):
    B, N0, _ = xyz0.shape
    N1 = xyz1.shape[1]
    S1 = xyz2.shape[1]
    NCH = 4
    H = B // NCH

    q1T = jnp.transpose(xyz1, (0, 2, 1))
    q0T = jnp.transpose(xyz0, (0, 2, 1))
    sel1 = _flatten_sel(_select_tc(q1T, xyz2, nblk=1024))
    t1 = jnp.transpose(x2, (0, 2, 1)).reshape(B * S1, -1)
    interp1 = _interp_sc(t1, sel1)

    sels = [_flatten_sel(_select_tc(q0T[:H], xyz1[:H], nblk=1024))]

    x1T = jnp.transpose(x1, (0, 2, 1))
    f1 = jnp.concatenate([x1T, interp1.reshape(B, N1, -1)], axis=2)
    t2 = f1.reshape(B * N1, -1)

    interps = [_interp_sc(t2, sels[0])]
    for c in range(1, NCH):
        lo = c * H
        sels.append(_flatten_sel(
            _select_tc(q0T[lo:lo + H], xyz1[lo:lo + H], nblk=1024, b_base=lo)))
        interps.append(_interp_sc(t2, sels[c]))
    out = _finalize_tc(x0, interps[0].reshape(H, N0, -1), nblk=2048, nb=H)
    for c in range(1, NCH):
        out = _finalize_tc(x0, interps[c].reshape(H, N0, -1), nblk=2048,
                           b_base=c * H, nb=H, out=out)
    return out


# speedup vs baseline: 1.0012x; 1.0012x over previous
import functools

import jax
import jax.numpy as jnp
from jax import lax
from jax.experimental import pallas as pl
from jax.experimental.pallas import tpu as pltpu
from jax.experimental.pallas import tpu_sc as plsc

_K = 3
_G = 16


def _splat(vec, isplat):
    dn = lax.GatherDimensionNumbers(
        offset_dims=(), collapsed_slice_dims=(0,), start_index_map=(0,))
    return lax.gather(vec, isplat[:, None], dn, (1,),
                      mode=lax.GatherScatterMode.PROMISE_IN_BOUNDS)


def _select_body(qT_ref, src_ref, sel_ref, *, S, b_base):
    b = pl.program_id(0) + b_base
    qf = qT_ref[0]
    sf = src_ref[0]
    qx = qf[0:1, :]
    qy = qf[1:2, :]
    qz = qf[2:3, :]
    sx = sf[:, 0:1]
    sy = sf[:, 1:2]
    sz = sf[:, 2:3]
    q2 = qx * qx + qy * qy + qz * qz
    s2 = sx * sx + sy * sy + sz * sz
    m = jnp.dot(sf, qf, preferred_element_type=jnp.float32,
                precision=lax.Precision.DEFAULT)
    d = -2.0 * m
    d = d + q2
    d = d + s2

    iota = lax.broadcasted_iota(jnp.int32, d.shape, 0)
    dd = d
    mins = []
    idxs = []
    for _ in range(_K):
        mk = jnp.min(dd, axis=0, keepdims=True)
        ik = jnp.min(jnp.where(dd <= mk, iota, S), axis=0, keepdims=True)
        dd = jnp.where(iota == ik, jnp.inf, dd)
        mins.append(mk)
        idxs.append(ik)

    recips = [1.0 / (m_ + 1e-8) for m_ in mins]
    norm = (recips[0] + recips[1]) + recips[2]
    gidx = jnp.concatenate(idxs, axis=0) + b * S
    wbits = lax.bitcast_convert_type(
        jnp.concatenate([r / norm for r in recips], axis=0), jnp.int32)
    sel_ref[0] = jnp.concatenate([gidx, wbits], axis=0)


def _select_tc(qT, src, *, nblk, b_base=0):
    B, _, N = qT.shape
    S = src.shape[1]
    grid = (B, N // nblk)
    return pl.pallas_call(
        functools.partial(_select_body, S=S, b_base=b_base),
        grid=grid,
        in_specs=[
            pl.BlockSpec((1, 3, nblk), lambda b, j: (b, 0, j)),
            pl.BlockSpec((1, S, 3), lambda b, j: (b, 0, 0)),
        ],
        out_specs=pl.BlockSpec((1, 6, nblk), lambda b, j: (b, 0, j)),
        out_shape=jax.ShapeDtypeStruct((B, 6, N), jnp.int32),
        compiler_params=pltpu.CompilerParams(
            dimension_semantics=("parallel", "arbitrary"),
        ),
    )(qT, src)


def _interp_sc(table, sel):
    R, D = table.shape
    BN = sel.shape[0] * _G
    info = plsc.get_sparse_core_info()
    NC, NS = info.num_cores, info.num_subcores
    NW = NC * NS
    C = BN // NW
    nsub = C // _G
    npair = nsub // 2
    mesh = plsc.VectorSubcoreMesh(core_axis_name="c", subcore_axis_name="s")

    @functools.partial(
        pl.kernel, mesh=mesh,
        out_type=jax.ShapeDtypeStruct((BN, D), jnp.float32),
        scratch_types=[
            pltpu.VMEM((6 * _G,), jnp.int32),
            pltpu.VMEM((6 * _G,), jnp.int32),
            pltpu.VMEM((_G, D), jnp.float32),
            pltpu.VMEM((_G, D), jnp.float32),
            pltpu.VMEM((_G, D), jnp.float32),
            pltpu.VMEM((_G, D), jnp.float32),
            pltpu.VMEM((_G, D), jnp.float32),
            pltpu.VMEM((_G, D), jnp.float32),
            pltpu.VMEM((_G, D), jnp.float32),
            pltpu.VMEM((_G, D), jnp.float32),
            pltpu.SemaphoreType.DMA,
            pltpu.SemaphoreType.DMA,
        ],
    )
    def k(t_hbm, sel_hbm, out_hbm,
          selA, selB, r0A, r1A, r2A, accA, r0B, r1B, r2B, accB, semA, semB):
        wid = lax.axis_index("s") * NC + lax.axis_index("c")
        base = wid * C
        bufA = (selA, (r0A, r1A, r2A), accA, semA)
        bufB = (selB, (r0B, r1B, r2B), accB, semB)

        def fire(c, buf):
            sel_v, rows, _, sem = buf
            cg = wid * nsub + c
            pltpu.sync_copy(sel_hbm.at[cg], sel_v)
            for kk in range(_K):
                idxk = sel_v[pl.ds(kk * _G, _G)]
                pltpu.async_copy(t_hbm.at[idxk], rows[kk], sem)

        def drain_compute_store(c, buf):
            sel_v, rows, acc, sem = buf
            goff = base + c * _G
            for kk in range(_K):
                idxk = sel_v[pl.ds(kk * _G, _G)]
                pltpu.make_async_copy(t_hbm.at[idxk], rows[kk], sem).wait()
            w0 = lax.bitcast_convert_type(sel_v[pl.ds(3 * _G, _G)], jnp.float32)
            w1 = lax.bitcast_convert_type(sel_v[pl.ds(4 * _G, _G)], jnp.float32)
            w2 = lax.bitcast_convert_type(sel_v[pl.ds(5 * _G, _G)], jnp.float32)
            r0, r1, r2 = rows

            def per_q(i, _):
                isplat = jnp.full((_G,), 0, jnp.int32) + i
                s0 = _splat(w0, isplat)
                s1 = _splat(w1, isplat)
                s2_ = _splat(w2, isplat)
                for v in range(D // _G):
                    sl = pl.ds(v * _G, _G)
                    acc[i, sl] = (r0[i, sl] * s0 + r1[i, sl] * s1) + r2[i, sl] * s2_
                return 0

            lax.fori_loop(0, _G, per_q, 0)
            pltpu.sync_copy(acc, out_hbm.at[pl.ds(goff, _G)])

        fire(0, bufA)

        def pair(p, _):
            cA = 2 * p
            fire(cA + 1, bufB)
            drain_compute_store(cA, bufA)

            @pl.when(cA + 2 < nsub)
            def _():
                fire(cA + 2, bufA)

            drain_compute_store(cA + 1, bufB)
            return 0

        lax.fori_loop(0, npair, pair, 0)

    return k(table, sel)


def _finalize_body(x0_ref, it_ref, *rest, D1):
    out_ref = rest[-1]
    out_ref[0, :D1, :] = x0_ref[0]
    out_ref[0, D1:, :] = jnp.transpose(it_ref[0], (1, 0))


def _finalize_tc(x0, interp, *, nblk, b_base=0, nb=None, out=None):
    B, D1, N = x0.shape
    D2 = interp.shape[2]
    nb = B if nb is None else nb
    args = [x0, interp]
    in_specs = [
        pl.BlockSpec((1, D1, nblk), lambda b, j: (b + b_base, 0, j)),
        pl.BlockSpec((1, nblk, D2), lambda b, j: (b, j, 0)),
    ]
    io_alias = {}
    if out is not None:
        args.append(out)
        in_specs.append(pl.BlockSpec(memory_space=pltpu.MemorySpace.HBM))
        io_alias = {2: 0}
    return pl.pallas_call(
        functools.partial(_finalize_body, D1=D1),
        grid=(nb, N // nblk),
        in_specs=in_specs,
        out_specs=pl.BlockSpec((1, D1 + D2, nblk),
                               lambda b, j: (b + b_base, 0, j)),
        out_shape=jax.ShapeDtypeStruct((B, D1 + D2, N), jnp.float32),
        input_output_aliases=io_alias,
        compiler_params=pltpu.CompilerParams(
            dimension_semantics=("parallel", "arbitrary"),
        ),
    )(*args)


def _flatten_sel(sel):
    B, _, N = sel.shape
    f = jnp.transpose(sel, (1, 0, 2)).reshape(6, (B * N) // _G, _G)
    return jnp.transpose(f, (1, 0, 2)).reshape((B * N) // _G, 6 * _G)


def kernel(xyz0, xyz1, xyz2, x0, x1, x2):
    B, N0, _ = xyz0.shape
    N1 = xyz1.shape[1]
    S1 = xyz2.shape[1]
    NCH = 4
    H = B // NCH

    q1T = jnp.transpose(xyz1, (0, 2, 1))
    q0T = jnp.transpose(xyz0, (0, 2, 1))
    sel1 = _flatten_sel(_select_tc(q1T, xyz2, nblk=1024))
    t1 = jnp.transpose(x2, (0, 2, 1)).reshape(B * S1, -1)
    interp1 = _interp_sc(t1, sel1)

    sels = [_flatten_sel(_select_tc(q0T[:H], xyz1[:H], nblk=1024))]

    x1T = jnp.transpose(x1, (0, 2, 1))
    f1 = jnp.concatenate([x1T, interp1.reshape(B, N1, -1)], axis=2)
    t2 = f1.reshape(B * N1, -1)

    interps = [_interp_sc(t2, sels[0])]
    for c in range(1, NCH):
        lo = c * H
        sels.append(_flatten_sel(
            _select_tc(q0T[lo:lo + H], xyz1[lo:lo + H], nblk=1024, b_base=lo)))
        interps.append(_interp_sc(t2, sels[c]))
    out = _finalize_tc(x0, interps[0].reshape(H, N0, -1), nblk=1024, nb=H)
    for c in range(1, NCH):
        out = _finalize_tc(x0, interps[c].reshape(H, N0, -1), nblk=1024,
                           b_base=c * H, nb=H, out=out)
    return out
